# baseline (speedup 1.0000x reference)
import functools

import jax
import jax.numpy as jnp
from jax import lax
from jax.experimental import pallas as pl
from jax.experimental.pallas import tpu as pltpu
from jax.experimental.pallas import tpu_sc as plsc


def _fps_kernel(n, s, cols, px_ref, py_ref, pz_ref, idx_ref, psx_ref, psy_ref,
                psz_ref):
    px = px_ref[...]
    py = py_ref[...]
    pz = pz_ref[...]
    flat = (jax.lax.broadcasted_iota(jnp.int32, (8, cols), 0) * cols +
            jax.lax.broadcasted_iota(jnp.int32, (8, cols), 1))
    valid = flat < n
    lane = jax.lax.broadcasted_iota(jnp.int32, (1, 128), 1)

    def write_slot(ref, i, val):
        r = i // 128
        c = i % 128
        row = ref[pl.ds(r, 1), :]
        ref[pl.ds(r, 1), :] = jnp.where(lane == c, val, row)

    def emit(i, nxt, cx, cy, cz):
        write_slot(idx_ref, i, nxt)
        write_slot(psx_ref, i, cx)
        write_slot(psy_ref, i, cy)
        write_slot(psz_ref, i, cz)

    eq0 = flat == 0
    cx = jnp.sum(jnp.where(eq0, px, 0.0))
    cy = jnp.sum(jnp.where(eq0, py, 0.0))
    cz = jnp.sum(jnp.where(eq0, pz, 0.0))
    dists = (px - cx) ** 2 + (py - cy) ** 2 + (pz - cz) ** 2
    dists = jnp.where(valid, dists, -jnp.inf)
    emit(0, jnp.int32(0), cx, cy, cz)

    def body(i, dists):
        m = jnp.max(dists)
        nxt = jnp.min(jnp.where(dists == m, flat, jnp.int32(2**30)))
        eq = flat == nxt
        cx = jnp.sum(jnp.where(eq, px, 0.0))
        cy = jnp.sum(jnp.where(eq, py, 0.0))
        cz = jnp.sum(jnp.where(eq, pz, 0.0))
        d = (px - cx) ** 2 + (py - cy) ** 2 + (pz - cz) ** 2
        dists = jnp.minimum(dists, d)
        emit(i, nxt, cx, cy, cz)
        return dists

    jax.lax.fori_loop(1, s, body, dists)


def _run_fps(pos, s):
    n = pos.shape[0]
    npad = ((n + 1023) // 1024) * 1024
    cols = npad // 8
    spad = ((s + 127) // 128) * 128
    p = jnp.pad(pos, ((0, npad - n), (0, 0)))
    px = p[:, 0].reshape(8, cols)
    py = p[:, 1].reshape(8, cols)
    pz = p[:, 2].reshape(8, cols)
    out_shape = (
        jax.ShapeDtypeStruct((spad // 128, 128), jnp.int32),
        jax.ShapeDtypeStruct((spad // 128, 128), jnp.float32),
        jax.ShapeDtypeStruct((spad // 128, 128), jnp.float32),
        jax.ShapeDtypeStruct((spad // 128, 128), jnp.float32),
    )
    idx, psx, psy, psz = pl.pallas_call(
        functools.partial(_fps_kernel, n, s, cols),
        out_shape=out_shape,
    )(px, py, pz)
    idx = idx.reshape(-1)[:s]
    pos_s = jnp.stack(
        [psx.reshape(-1)[:s], psy.reshape(-1)[:s], psz.reshape(-1)[:s]],
        axis=1)
    return idx, pos_s


def _mlp_kernel(bq, k, h_ref, w1_ref, b1_ref, w2_ref, b2_ref, w3_ref, b3_ref,
                out_ref):
    h = h_ref[...]
    dot = functools.partial(
        jnp.dot, preferred_element_type=jnp.float32,
        precision=jax.lax.Precision.HIGHEST)
    h = jnp.maximum(dot(h, w1_ref[...]) + b1_ref[...], 0.0)
    h = jnp.maximum(dot(h, w2_ref[...]) + b2_ref[...], 0.0)
    h = jnp.maximum(dot(h, w3_ref[...]) + b3_ref[...], 0.0)
    co = h.shape[-1]
    out_ref[...] = jnp.max(h.reshape(bq, k, co), axis=1)


def _run_mlp(hrows, s, k, w1, b1, w2, b2, w3, b3):
    ci = hrows.shape[-1]
    c1 = w1.shape[1]
    c2 = w2.shape[1]
    c3 = w3.shape[1]
    bq = 128
    spad = ((s + bq - 1) // bq) * bq
    if spad != s:
        hrows = jnp.pad(hrows, ((0, (spad - s) * k), (0, 0)))
    grid = (spad // bq,)
    return pl.pallas_call(
        functools.partial(_mlp_kernel, bq, k),
        grid=grid,
        in_specs=[
            pl.BlockSpec((bq * k, ci), lambda i: (i, 0)),
            pl.BlockSpec((ci, c1), lambda i: (0, 0)),
            pl.BlockSpec((1, c1), lambda i: (0, 0)),
            pl.BlockSpec((c1, c2), lambda i: (0, 0)),
            pl.BlockSpec((1, c2), lambda i: (0, 0)),
            pl.BlockSpec((c2, c3), lambda i: (0, 0)),
            pl.BlockSpec((1, c3), lambda i: (0, 0)),
        ],
        out_specs=pl.BlockSpec((bq, c3), lambda i: (i, 0)),
        out_shape=jax.ShapeDtypeStruct((spad, c3), jnp.float32),
    )(hrows, w1, b1.reshape(1, c1), w2, b2.reshape(1, c2), w3,
      b3.reshape(1, c3))[:s]


_CAND = 768


def _sc_radius_filter(d2m, sidx, rr):
    npad = d2m.shape[1]
    spad = d2m.shape[0]
    nw = 32
    rows_pw = spad // nw
    cap = _CAND + 16
    mesh = plsc.VectorSubcoreMesh(core_axis_name="c", subcore_axis_name="s")

    import functools as _ft

    @_ft.partial(
        pl.kernel, mesh=mesh,
        compiler_params=pltpu.CompilerParams(needs_layout_passes=False),
        out_type=(
            jax.ShapeDtypeStruct((spad, _CAND), jnp.float32),
            jax.ShapeDtypeStruct((spad, _CAND), jnp.int32),
        ),
        scratch_types=[
            pltpu.VMEM((npad,), jnp.float32),
            pltpu.VMEM((rows_pw + 16,), jnp.int32),
            pltpu.VMEM((cap,), jnp.float32),
            pltpu.VMEM((cap,), jnp.int32),
        ],
    )
    def filt(d2h, sih, cd2_out, cidx_out, row_v, si_v, bufd, bufi):
        wid = lax.axis_index("s") * 2 + lax.axis_index("c")
        base_row = wid * rows_pw
        pltpu.sync_copy(sih.at[pl.ds(base_row, rows_pw)],
                        si_v.at[pl.ds(0, rows_pw)])
        lane = lax.iota(jnp.int32, 16)

        def row_body(r, _):
            rg = base_row + r
            pltpu.sync_copy(d2h.at[rg], row_v)
            self_i = si_v[pl.ds(r, 16)][0]

            def pre(j, _):
                bufd[pl.ds(j * 16, 16)] = jnp.full((16,), jnp.inf,
                                                   jnp.float32)
                bufi[pl.ds(j * 16, 16)] = jnp.full((16,), 0,
                                                   jnp.int32) + self_i
                return 0

            lax.fori_loop(0, cap // 16, pre, 0, unroll=4)

            def fbody(c, offset):
                b = c * 16
                dv = row_v[pl.ds(b, 16)]
                m = dv <= rr
                ones = jnp.where(m, jnp.full((16,), 1, jnp.int32),
                                 jnp.full((16,), 0, jnp.int32))
                cnt = plsc.cumsum(ones)[15]
                o = jnp.minimum(offset, _CAND)
                plsc.store_compressed(bufd.at[pl.ds(o, 16)], dv, mask=m)
                plsc.store_compressed(bufi.at[pl.ds(o, 16)], b + lane,
                                      mask=m)
                return offset + cnt

            lax.fori_loop(0, npad // 16, fbody, 0, unroll=2)
            pltpu.sync_copy(bufd.at[pl.ds(0, _CAND)], cd2_out.at[rg])
            pltpu.sync_copy(bufi.at[pl.ds(0, _CAND)], cidx_out.at[rg])
            return 0

        lax.fori_loop(0, rows_pw, row_body, 0)

    return filt(d2m, sidx)


def kernel(x, pos, batch, W1, b1, W2, b2, W3, b3):
    n, d = x.shape
    s = int(n * 0.25)
    k = 64
    r = 0.2

    idx, pos_s = _run_fps(pos, s)

    n_pad = ((n + 1023) // 1024) * 1024
    s_pad = ((s + 255) // 256) * 256
    pos_q = jnp.pad(pos_s, ((0, s_pad - s), (0, 0)), constant_values=1e9)
    pos_a = jnp.pad(pos, ((0, n_pad - n), (0, 0)), constant_values=1e9)
    qq = jnp.sum(pos_q ** 2, axis=1, keepdims=True)
    pp = jnp.sum(pos_a ** 2, axis=1)[None, :]
    d2m = qq + pp - 2.0 * (pos_q @ pos_a.T)
    sidx = jnp.pad(idx, (0, s_pad - s))
    cd2, cidx = _sc_radius_filter(d2m, sidx, jnp.float32(r * r))
    cd2 = cd2[:s]
    cidx = cidx[:s]
    _, tpos = jax.lax.top_k(-cd2, k)
    nbr = jnp.take_along_axis(cidx, tpos, axis=1)

    xj = x[nbr]
    rel = pos[nbr] - pos_s[:, None, :]
    hrows = jnp.concatenate([xj, rel], axis=-1).reshape(s * k, d + 3)

    out = _run_mlp(hrows, s, k, W1, b1, W2, b2, W3, b3)
    return (out, pos_s, batch[idx])
